# Initial kernel scaffold; baseline (speedup 1.0000x reference)
import jax
import jax.numpy as jnp
from jax.experimental import pallas as pl


def kernel(index, weight):
    raise NotImplementedError("write your pallas kernel here")


# speedup vs baseline: 5.2437x; 5.2437x over previous
import functools

import jax
import jax.numpy as jnp
import numpy as np
from jax import lax
from jax.experimental import pallas as pl
from jax.experimental.pallas import tpu as pltpu
from jax.experimental.pallas import tpu_sc as plsc

NC, NS, L = 2, 16, 16
NW = NC * NS
ROWS, COLS = 4096, 200
B = ROWS * COLS
PER_W = B // NW
CHUNK = 1024
NCHUNK = PER_W // CHUNK
HID = 32
NBUF = 2
EMBED_NUM = 100000
HALF = EMBED_NUM // 2

_mesh = plsc.VectorSubcoreMesh(core_axis_name="c", subcore_axis_name="s")


@functools.partial(
    pl.kernel,
    out_type=jax.ShapeDtypeStruct((B, HID), jnp.float32),
    mesh=_mesh,
    scratch_types=[
        pltpu.VMEM((PER_W,), jnp.float32),
        pltpu.VMEM((PER_W,), jnp.int32),
        pltpu.VMEM((NBUF, CHUNK, HID), jnp.float32),
        [pltpu.SemaphoreType.DMA] * NBUF,
    ],
    compiler_params=pltpu.CompilerParams(use_tc_tiling_on_sc=False),
)
def _embed(ang_hbm, table_hbm, out_hbm, ang_v, idx_v, rows_v, sems):
    wid = lax.axis_index("s") * NC + lax.axis_index("c")
    base = wid * PER_W
    pltpu.sync_copy(ang_hbm.at[pl.ds(base, PER_W)], ang_v)

    pi = jnp.float32(np.pi)

    def quant(i, carry):
        x = ang_v[pl.ds(i * L, L)]
        v = (x / pi + 1.0) * jnp.float32(HALF)
        v = jnp.minimum(jnp.maximum(v, 0.0), jnp.float32(EMBED_NUM - 1))
        idx_v[pl.ds(i * L, L)] = v.astype(jnp.int32)
        return carry

    lax.fori_loop(0, PER_W // L, quant, 0)

    def fire(k):
        return pltpu.async_copy(
            table_hbm.at[idx_v.at[pl.ds(k * CHUNK, CHUNK)]],
            rows_v.at[k % NBUF],
            sems[k % NBUF],
        )

    copies = [fire(0), fire(1)]
    for k in range(NCHUNK):
        copies[k % NBUF].wait()
        pltpu.sync_copy(rows_v.at[k % NBUF],
                        out_hbm.at[pl.ds(base + k * CHUNK, CHUNK)])
        if k + NBUF < NCHUNK:
            copies[k % NBUF] = fire(k + NBUF)


def kernel(index, weight):
    out = _embed(index.reshape(B), weight)
    return out.reshape(ROWS, COLS, HID)


# speedup vs baseline: 5.2612x; 1.0033x over previous
import functools

import jax
import jax.numpy as jnp
import numpy as np
from jax import lax
from jax.experimental import pallas as pl
from jax.experimental.pallas import tpu as pltpu
from jax.experimental.pallas import tpu_sc as plsc

NC, NS, L = 2, 16, 16
NW = NC * NS
ROWS, COLS = 4096, 200
B = ROWS * COLS
PER_W = B // NW
CHUNK = 512
NCHUNK = PER_W // CHUNK
HID = 32
NBUF = 4
EMBED_NUM = 100000
HALF = EMBED_NUM // 2

_mesh = plsc.VectorSubcoreMesh(core_axis_name="c", subcore_axis_name="s")


@functools.partial(
    pl.kernel,
    out_type=jax.ShapeDtypeStruct((B, HID), jnp.float32),
    mesh=_mesh,
    scratch_types=[
        pltpu.VMEM((PER_W,), jnp.float32),
        pltpu.VMEM((PER_W,), jnp.int32),
        pltpu.VMEM((NBUF, CHUNK, HID), jnp.float32),
        [pltpu.SemaphoreType.DMA] * NBUF,
        [pltpu.SemaphoreType.DMA] * NBUF,
    ],
    compiler_params=pltpu.CompilerParams(use_tc_tiling_on_sc=False),
)
def _embed(ang_hbm, table_hbm, out_hbm, ang_v, idx_v, rows_v, gsems, wsems):
    wid = lax.axis_index("s") * NC + lax.axis_index("c")
    base = wid * PER_W
    pltpu.sync_copy(ang_hbm.at[pl.ds(base, PER_W)], ang_v)

    pi = jnp.float32(np.pi)

    def quant(i, carry):
        x = ang_v[pl.ds(i * L, L)]
        v = (x / pi + 1.0) * jnp.float32(HALF)
        v = jnp.minimum(jnp.maximum(v, 0.0), jnp.float32(EMBED_NUM - 1))
        idx_v[pl.ds(i * L, L)] = v.astype(jnp.int32)
        return carry

    lax.fori_loop(0, PER_W // L, quant, 0)

    def fire_gather(k):
        return pltpu.async_copy(
            table_hbm.at[idx_v.at[pl.ds(k * CHUNK, CHUNK)]],
            rows_v.at[k % NBUF],
            gsems[k % NBUF],
        )

    def fire_writeback(k):
        return pltpu.async_copy(
            rows_v.at[k % NBUF],
            out_hbm.at[pl.ds(base + k * CHUNK, CHUNK)],
            wsems[k % NBUF],
        )

    gcopies = [fire_gather(k) for k in range(NBUF)]
    wcopies = [None] * NBUF
    for k in range(NCHUNK):
        b = k % NBUF
        gcopies[b].wait()
        wcopies[b] = fire_writeback(k)
        if k + NBUF < NCHUNK:
            wcopies[b].wait()
            gcopies[b] = fire_gather(k + NBUF)
    for k in range(NCHUNK - NBUF, NCHUNK):
        wcopies[k % NBUF].wait()


def kernel(index, weight):
    out = _embed(index.reshape(B), weight)
    return out.reshape(ROWS, COLS, HID)
